# baseline (speedup 1.0000x reference)
import jax
import jax.numpy as jnp
from jax import lax
from jax.experimental import pallas as pl
from jax.experimental.pallas import tpu as pltpu
from jax.experimental.pallas import tpu_sc as plsc

T, G, N, E = 4, 2, 10000, 320000
D = 128
P = T * G
NC, NS = 2, 16
NW = NC * NS
EPW = E // NW
K = 80
NCHUNK = EPW // K


def _gather_body(table, src, out, src_v, rows_v, sem):
  c = lax.axis_index("c")
  s = lax.axis_index("s")
  wbase = (c * NS + s) * EPW

  def pair_body(p, _):
    def chunk_body(i, _):
      base = p * E + wbase + i * K
      pltpu.sync_copy(src.at[pl.ds(base, K)], src_v)
      pltpu.async_copy(table.at[src_v], rows_v, sem).wait()
      pltpu.sync_copy(rows_v, out.at[pl.ds(base, K)])
      return 0

    lax.fori_loop(0, NCHUNK, chunk_body, 0)
    return 0

  lax.fori_loop(0, P, pair_body, 0)


_gather_sc = pl.kernel(
    _gather_body,
    out_type=jax.ShapeDtypeStruct((P * E, D), jnp.float32),
    mesh=plsc.VectorSubcoreMesh(core_axis_name="c", subcore_axis_name="s",
                                num_cores=NC, num_subcores=NS),
    scratch_types=[
        pltpu.VMEM((K,), jnp.int32),
        pltpu.VMEM((K, D), jnp.float32),
        pltpu.SemaphoreType.DMA,
    ],
)


BN = 1000


def _sage_dense_body(ps_ref, cp_ref, h_ref, wl_ref, wr_ref, b_ref, o_ref):
  s_sum = ps_ref[0]
  cnt16 = jnp.sum(cp_ref[0], axis=-1, keepdims=True)
  inv = 16.0 / jnp.maximum(cnt16, 16.0)
  mean = s_sum * inv
  h = h_ref[0]
  y = (lax.dot_general(mean, wl_ref[0], (((1,), (1,)), ((), ())),
                       preferred_element_type=jnp.float32)
       + lax.dot_general(h, wr_ref[0], (((1,), (1,)), ((), ())),
                         preferred_element_type=jnp.float32)
       + b_ref[0])
  o_ref[0] = jnp.tanh(y)


def _sage_dense(ps, cp, h, wl, wr, b):
  grid = (P, N // BN)
  return pl.pallas_call(
      _sage_dense_body,
      grid=grid,
      in_specs=[
          pl.BlockSpec((1, BN, D), lambda p, nb: (p, nb, 0)),
          pl.BlockSpec((1, BN, 16), lambda p, nb: (p, nb, 0)),
          pl.BlockSpec((1, BN, D), lambda p, nb: (p, nb, 0)),
          pl.BlockSpec((1, D, D), lambda p, nb: (p % G, 0, 0)),
          pl.BlockSpec((1, D, D), lambda p, nb: (p % G, 0, 0)),
          pl.BlockSpec((1, 1, D), lambda p, nb: (p % G, 0, 0)),
      ],
      out_specs=pl.BlockSpec((1, BN, D), lambda p, nb: (p, nb, 0)),
      out_shape=jax.ShapeDtypeStruct((P, N, D), jnp.float32),
  )(ps, cp, h, wl, wr, b.reshape(G, 1, D))


def _lstm_body(x0_ref, x1_ref, wih_ref, whh_ref, b_ref, o_ref):
  x0 = x0_ref[0]
  x1 = x1_ref[0]
  wih = wih_ref[...]
  whh = whh_ref[...]
  b = b_ref[...]
  g0 = lax.dot_general(x0, wih, (((1,), (1,)), ((), ())),
                       preferred_element_type=jnp.float32) + b
  i0 = jax.nn.sigmoid(g0[:, 0:D])
  gg0 = jnp.tanh(g0[:, 2 * D:3 * D])
  o0 = jax.nn.sigmoid(g0[:, 3 * D:4 * D])
  c1 = i0 * gg0
  h1 = o0 * jnp.tanh(c1)
  g1 = (lax.dot_general(x1, wih, (((1,), (1,)), ((), ())),
                        preferred_element_type=jnp.float32)
        + lax.dot_general(h1, whh, (((1,), (1,)), ((), ())),
                          preferred_element_type=jnp.float32) + b)
  i1 = jax.nn.sigmoid(g1[:, 0:D])
  f1 = jax.nn.sigmoid(g1[:, D:2 * D])
  gg1 = jnp.tanh(g1[:, 2 * D:3 * D])
  o1 = jax.nn.sigmoid(g1[:, 3 * D:4 * D])
  c2 = f1 * c1 + i1 * gg1
  o_ref[...] = o1 * jnp.tanh(c2)


def _lstm(h2, wih, whh, bsum):
  nb_per_t = N // BN
  grid = (T * nb_per_t,)
  return pl.pallas_call(
      _lstm_body,
      grid=grid,
      in_specs=[
          pl.BlockSpec((1, BN, D), lambda r: (G * (r // nb_per_t),
                                              r % nb_per_t, 0)),
          pl.BlockSpec((1, BN, D), lambda r: (G * (r // nb_per_t) + 1,
                                              r % nb_per_t, 0)),
          pl.BlockSpec((4 * D, D), lambda r: (0, 0)),
          pl.BlockSpec((4 * D, D), lambda r: (0, 0)),
          pl.BlockSpec((1, 4 * D), lambda r: (0, 0)),
      ],
      out_specs=pl.BlockSpec((BN, D), lambda r: (r, 0)),
      out_shape=jax.ShapeDtypeStruct((T * N, D), jnp.float32),
  )(h2, h2, wih, whh, bsum)


@jax.jit
def kernel(x, edge_index, Wl1, Wr1, b1, Wl2, Wr2, b2, W_ih, W_hh, b_ih,
           b_hh):
  ei = edge_index.astype(jnp.int32)
  src = ei[:, :, 0, :].reshape(P, E)
  dst = ei[:, :, 1, :].reshape(P, E)
  offs = (jnp.arange(P, dtype=jnp.int32) * N)[:, None]
  src_flat = (src + offs).reshape(P * E)
  xt = x.reshape(P, N, D)

  def seg_sum(msgs):
    def one(mp, dp):
      return jax.ops.segment_sum(mp, dp, num_segments=N)
    return jax.vmap(one)(msgs.reshape(P, E, D), dst)

  cnt = jax.vmap(lambda dp: jax.ops.segment_sum(
      jnp.ones((E,), jnp.float32), dp, num_segments=N))(dst)
  cnt16 = jnp.broadcast_to(cnt[:, :, None], (P, N, 16))

  msgs1 = _gather_sc(xt.reshape(P * N, D), src_flat)
  h1 = _sage_dense(seg_sum(msgs1), cnt16, xt, Wl1, Wr1, b1)

  msgs2 = _gather_sc(h1.reshape(P * N, D), src_flat)
  h2 = _sage_dense(seg_sum(msgs2), cnt16, h1, Wl2, Wr2, b2)

  bsum = (b_ih + b_hh).reshape(1, 4 * D)
  return _lstm(h2, W_ih, W_hh, bsum)


# speedup vs baseline: 1.0381x; 1.0381x over previous
import jax
import jax.numpy as jnp
from jax import lax
from jax.experimental import pallas as pl
from jax.experimental.pallas import tpu as pltpu
from jax.experimental.pallas import tpu_sc as plsc

T, G, N, E = 4, 2, 10000, 320000
D = 128
P = T * G
NC, NS = 2, 16
NW = NC * NS
EPW = E // NW
K = 80
NCHUNK = EPW // K


def _gather_body(table, src, out, src_v0, src_v1, rows_v0, rows_v1, sem0,
                 sem1):
  c = lax.axis_index("c")
  s = lax.axis_index("s")
  wbase = (c * NS + s) * EPW

  def pair_body(p, _):
    def chunk_body(i, _):
      b0 = p * E + wbase + (2 * i) * K
      b1 = b0 + K
      pltpu.sync_copy(src.at[pl.ds(b0, K)], src_v0)
      d0 = pltpu.async_copy(table.at[src_v0], rows_v0, sem0)
      pltpu.sync_copy(src.at[pl.ds(b1, K)], src_v1)
      d1 = pltpu.async_copy(table.at[src_v1], rows_v1, sem1)
      d0.wait()
      pltpu.sync_copy(rows_v0, out.at[pl.ds(b0, K)])
      d1.wait()
      pltpu.sync_copy(rows_v1, out.at[pl.ds(b1, K)])
      return 0

    lax.fori_loop(0, NCHUNK // 2, chunk_body, 0)

    bt = p * E + wbase + (NCHUNK - 1) * K
    pltpu.sync_copy(src.at[pl.ds(bt, K)], src_v0)
    pltpu.async_copy(table.at[src_v0], rows_v0, sem0).wait()
    pltpu.sync_copy(rows_v0, out.at[pl.ds(bt, K)])
    return 0

  lax.fori_loop(0, P, pair_body, 0)


_gather_sc = pl.kernel(
    _gather_body,
    out_type=jax.ShapeDtypeStruct((P * E, D), jnp.float32),
    mesh=plsc.VectorSubcoreMesh(core_axis_name="c", subcore_axis_name="s",
                                num_cores=NC, num_subcores=NS),
    scratch_types=[
        pltpu.VMEM((K,), jnp.int32),
        pltpu.VMEM((K,), jnp.int32),
        pltpu.VMEM((K, D), jnp.float32),
        pltpu.VMEM((K, D), jnp.float32),
        pltpu.SemaphoreType.DMA,
        pltpu.SemaphoreType.DMA,
    ],
)


BN = 1000


def _sage_dense_body(ps_ref, cp_ref, h_ref, wl_ref, wr_ref, b_ref, o_ref):
  s_sum = ps_ref[0]
  cnt16 = jnp.sum(cp_ref[0], axis=-1, keepdims=True)
  inv = 16.0 / jnp.maximum(cnt16, 16.0)
  mean = s_sum * inv
  h = h_ref[0]
  y = (lax.dot_general(mean, wl_ref[0], (((1,), (1,)), ((), ())),
                       preferred_element_type=jnp.float32)
       + lax.dot_general(h, wr_ref[0], (((1,), (1,)), ((), ())),
                         preferred_element_type=jnp.float32)
       + b_ref[0])
  o_ref[0] = jnp.tanh(y)


def _sage_dense(ps, cp, h, wl, wr, b):
  grid = (P, N // BN)
  return pl.pallas_call(
      _sage_dense_body,
      grid=grid,
      in_specs=[
          pl.BlockSpec((1, BN, D), lambda p, nb: (p, nb, 0)),
          pl.BlockSpec((1, BN, 16), lambda p, nb: (p, nb, 0)),
          pl.BlockSpec((1, BN, D), lambda p, nb: (p, nb, 0)),
          pl.BlockSpec((1, D, D), lambda p, nb: (p % G, 0, 0)),
          pl.BlockSpec((1, D, D), lambda p, nb: (p % G, 0, 0)),
          pl.BlockSpec((1, 1, D), lambda p, nb: (p % G, 0, 0)),
      ],
      out_specs=pl.BlockSpec((1, BN, D), lambda p, nb: (p, nb, 0)),
      out_shape=jax.ShapeDtypeStruct((P, N, D), jnp.float32),
  )(ps, cp, h, wl, wr, b.reshape(G, 1, D))


def _lstm_body(x0_ref, x1_ref, wih_ref, whh_ref, b_ref, o_ref):
  x0 = x0_ref[0]
  x1 = x1_ref[0]
  wih = wih_ref[...]
  whh = whh_ref[...]
  b = b_ref[...]
  g0 = lax.dot_general(x0, wih, (((1,), (1,)), ((), ())),
                       preferred_element_type=jnp.float32) + b
  i0 = jax.nn.sigmoid(g0[:, 0:D])
  gg0 = jnp.tanh(g0[:, 2 * D:3 * D])
  o0 = jax.nn.sigmoid(g0[:, 3 * D:4 * D])
  c1 = i0 * gg0
  h1 = o0 * jnp.tanh(c1)
  g1 = (lax.dot_general(x1, wih, (((1,), (1,)), ((), ())),
                        preferred_element_type=jnp.float32)
        + lax.dot_general(h1, whh, (((1,), (1,)), ((), ())),
                          preferred_element_type=jnp.float32) + b)
  i1 = jax.nn.sigmoid(g1[:, 0:D])
  f1 = jax.nn.sigmoid(g1[:, D:2 * D])
  gg1 = jnp.tanh(g1[:, 2 * D:3 * D])
  o1 = jax.nn.sigmoid(g1[:, 3 * D:4 * D])
  c2 = f1 * c1 + i1 * gg1
  o_ref[...] = o1 * jnp.tanh(c2)


def _lstm(h2, wih, whh, bsum):
  nb_per_t = N // BN
  grid = (T * nb_per_t,)
  return pl.pallas_call(
      _lstm_body,
      grid=grid,
      in_specs=[
          pl.BlockSpec((1, BN, D), lambda r: (G * (r // nb_per_t),
                                              r % nb_per_t, 0)),
          pl.BlockSpec((1, BN, D), lambda r: (G * (r // nb_per_t) + 1,
                                              r % nb_per_t, 0)),
          pl.BlockSpec((4 * D, D), lambda r: (0, 0)),
          pl.BlockSpec((4 * D, D), lambda r: (0, 0)),
          pl.BlockSpec((1, 4 * D), lambda r: (0, 0)),
      ],
      out_specs=pl.BlockSpec((BN, D), lambda r: (r, 0)),
      out_shape=jax.ShapeDtypeStruct((T * N, D), jnp.float32),
  )(h2, h2, wih, whh, bsum)


@jax.jit
def kernel(x, edge_index, Wl1, Wr1, b1, Wl2, Wr2, b2, W_ih, W_hh, b_ih,
           b_hh):
  ei = edge_index.astype(jnp.int32)
  src = ei[:, :, 0, :].reshape(P, E)
  dst = ei[:, :, 1, :].reshape(P, E)
  offs = (jnp.arange(P, dtype=jnp.int32) * N)[:, None]
  src_flat = (src + offs).reshape(P * E)
  xt = x.reshape(P, N, D)

  def seg_sum(msgs):
    def one(mp, dp):
      return jax.ops.segment_sum(mp, dp, num_segments=N)
    return jax.vmap(one)(msgs.reshape(P, E, D), dst)

  cnt = jax.vmap(lambda dp: jax.ops.segment_sum(
      jnp.ones((E,), jnp.float32), dp, num_segments=N))(dst)
  cnt16 = jnp.broadcast_to(cnt[:, :, None], (P, N, 16))

  msgs1 = _gather_sc(xt.reshape(P * N, D), src_flat)
  h1 = _sage_dense(seg_sum(msgs1), cnt16, xt, Wl1, Wr1, b1)

  msgs2 = _gather_sc(h1.reshape(P * N, D), src_flat)
  h2 = _sage_dense(seg_sum(msgs2), cnt16, h1, Wl2, Wr2, b2)

  bsum = (b_ih + b_hh).reshape(1, 4 * D)
  return _lstm(h2, W_ih, W_hh, bsum)
